# baseline (speedup 1.0000x reference)
import jax
import jax.numpy as jnp
from jax.experimental import pallas as pl

N = 4096
K = 4096
FUSED = 1024
MLP_BLK = 512


def _mlp_body(
    x1_ref, x2_ref, wa1_ref, ba1_ref, wb1_ref, bb1_ref,
    wa2_ref, ba2_ref, wb2_ref, bb2_ref, o1_ref, o2_ref,
):
    for x_ref, wa, ba, wb, bb, o_ref in (
        (x1_ref, wa1_ref, ba1_ref, wb1_ref, bb1_ref, o1_ref),
        (x2_ref, wa2_ref, ba2_ref, wb2_ref, bb2_ref, o2_ref),
    ):
        h = jnp.maximum(
            jnp.dot(x_ref[...], wa[...], preferred_element_type=jnp.float32)
            + ba[...],
            0.0,
        ).astype(jnp.bfloat16)
        o_ref[...] = (
            jnp.dot(h, wb[...], preferred_element_type=jnp.float32) + bb[...]
        )


def _mlp_heads(x1, x2, W1a, b1a, W1b, b1b, W2a, b2a, W2b, b2b):
    H = W1a.shape[1]
    O = W1b.shape[1]
    xspec = pl.BlockSpec((MLP_BLK, FUSED), lambda i: (i, 0))
    wa_spec = pl.BlockSpec((FUSED, H), lambda i: (0, 0))
    ba_spec = pl.BlockSpec((1, H), lambda i: (0, 0))
    wb_spec = pl.BlockSpec((H, O), lambda i: (0, 0))
    bb_spec = pl.BlockSpec((1, O), lambda i: (0, 0))
    ospec = pl.BlockSpec((MLP_BLK, O), lambda i: (i, 0))
    out_sh = jax.ShapeDtypeStruct((N, O), jnp.float32)
    return pl.pallas_call(
        _mlp_body,
        grid=(N // MLP_BLK,),
        in_specs=[xspec, xspec, wa_spec, ba_spec, wb_spec, bb_spec,
                  wa_spec, ba_spec, wb_spec, bb_spec],
        out_specs=[ospec, ospec],
        out_shape=[out_sh, out_sh],
    )(
        x1.astype(jnp.bfloat16), x2.astype(jnp.bfloat16),
        W1a.astype(jnp.bfloat16), b1a.reshape(1, H),
        W1b.astype(jnp.bfloat16), b1b.reshape(1, O),
        W2a.astype(jnp.bfloat16), b2a.reshape(1, H),
        W2b.astype(jnp.bfloat16), b2b.reshape(1, O),
    )


def _ent_body(x_ref, s_ref, o_ref):
    ax = jnp.abs(x_ref[...])
    p = ax / s_ref[...]
    t = p * jnp.log(p + 1e-08)
    P = t[0:128, :]
    for k in range(1, 32):
        P = P + t[128 * k:128 * (k + 1), :]
    Q = P[0:8, :]
    for u in range(1, 16):
        Q = Q + P[8 * u:8 * (u + 1), :]
    Q = Q[0:4, :] + Q[4:8, :]
    Q = Q[0:2, :] + Q[2:4, :]
    o_ref[...] = -(Q[0:1, :] + Q[1:2, :])


def _pallas_e1(a, b, s1v):
    BLK = 512
    sa = s1v[:, :K]
    sb = s1v[:, K:]
    ea, eb = pl.pallas_call(
        _ent_body_pair,
        grid=(K // BLK,),
        in_specs=[
            pl.BlockSpec((N, BLK), lambda i: (0, i)),
            pl.BlockSpec((N, BLK), lambda i: (0, i)),
            pl.BlockSpec((1, BLK), lambda i: (0, i)),
            pl.BlockSpec((1, BLK), lambda i: (0, i)),
        ],
        out_specs=[
            pl.BlockSpec((1, BLK), lambda i: (0, i)),
            pl.BlockSpec((1, BLK), lambda i: (0, i)),
        ],
        out_shape=[
            jax.ShapeDtypeStruct((1, K), jnp.float32),
            jax.ShapeDtypeStruct((1, K), jnp.float32),
        ],
    )(a, b, sa, sb)
    return jnp.concatenate([ea[0], eb[0]])


def _ent_body_pair(a_ref, b_ref, sa_ref, sb_ref, oa_ref, ob_ref):
    _ent_body(a_ref, sa_ref, oa_ref)
    _ent_body(b_ref, sb_ref, ob_ref)


def _select_topk_by_entropy(x, fused_dim=FUSED):
    abs_x = jnp.abs(x)
    probs = abs_x / (abs_x.sum(axis=0, keepdims=True) + 1e-08)
    entropy = -(probs * jnp.log(probs + 1e-08)).sum(axis=0)
    _, topk_idx = jax.lax.top_k(entropy, fused_dim)
    return jnp.take(x, topk_idx, axis=1)


def kernel(a, b, W1a, b1a, W1b, b1b, W2a, b2a, W2b, b2b):
    S1 = jnp.concatenate([a, b], axis=1)
    s1v = jnp.abs(S1).sum(axis=0, keepdims=True) + 1e-08
    e1 = _pallas_e1(a, b, s1v)
    _, idx1 = jax.lax.top_k(e1, FUSED)
    fused1 = jnp.take(S1, idx1, axis=1)
    S2 = jnp.concatenate([fused1, b], axis=1)
    fused2 = _select_topk_by_entropy(S2)
    logits1, logits2 = _mlp_heads(
        fused1, fused2, W1a, b1a, W1b, b1b, W2a, b2a, W2b, b2b
    )
    return (logits1, logits2, fused1, fused2)


# speedup vs baseline: 1.0709x; 1.0709x over previous
import jax
import jax.numpy as jnp
from jax.experimental import pallas as pl

N = 4096
K = 4096
FUSED = 1024
MLP_BLK = 512


def _mlp_body(
    x1_ref, x2_ref, wa1_ref, ba1_ref, wb1_ref, bb1_ref,
    wa2_ref, ba2_ref, wb2_ref, bb2_ref, o1_ref, o2_ref,
):
    for x_ref, wa, ba, wb, bb, o_ref in (
        (x1_ref, wa1_ref, ba1_ref, wb1_ref, bb1_ref, o1_ref),
        (x2_ref, wa2_ref, ba2_ref, wb2_ref, bb2_ref, o2_ref),
    ):
        xb = x_ref[...].astype(jnp.bfloat16)
        h = jnp.maximum(
            jnp.dot(xb, wa[...], preferred_element_type=jnp.float32) + ba[...],
            0.0,
        ).astype(jnp.bfloat16)
        o_ref[...] = (
            jnp.dot(h, wb[...], preferred_element_type=jnp.float32) + bb[...]
        )


def _mlp_heads(x1, x2, W1a, b1a, W1b, b1b, W2a, b2a, W2b, b2b):
    H = W1a.shape[1]
    O = W1b.shape[1]
    xspec = pl.BlockSpec((MLP_BLK, FUSED), lambda i: (i, 0))
    wa_spec = pl.BlockSpec((FUSED, H), lambda i: (0, 0))
    ba_spec = pl.BlockSpec((1, H), lambda i: (0, 0))
    wb_spec = pl.BlockSpec((H, O), lambda i: (0, 0))
    bb_spec = pl.BlockSpec((1, O), lambda i: (0, 0))
    ospec = pl.BlockSpec((MLP_BLK, O), lambda i: (i, 0))
    out_sh = jax.ShapeDtypeStruct((N, O), jnp.float32)
    return pl.pallas_call(
        _mlp_body,
        grid=(N // MLP_BLK,),
        in_specs=[xspec, xspec, wa_spec, ba_spec, wb_spec, bb_spec,
                  wa_spec, ba_spec, wb_spec, bb_spec],
        out_specs=[ospec, ospec],
        out_shape=[out_sh, out_sh],
    )(
        x1, x2,
        W1a.astype(jnp.bfloat16), b1a.reshape(1, H),
        W1b.astype(jnp.bfloat16), b1b.reshape(1, O),
        W2a.astype(jnp.bfloat16), b2a.reshape(1, H),
        W2b.astype(jnp.bfloat16), b2b.reshape(1, O),
    )


def _select_topk_by_entropy(x, fused_dim=FUSED):
    abs_x = jnp.abs(x)
    probs = abs_x / (abs_x.sum(axis=0, keepdims=True) + 1e-08)
    entropy = -(probs * jnp.log(probs + 1e-08)).sum(axis=0)
    _, topk_idx = jax.lax.top_k(entropy, fused_dim)
    return jnp.take(x, topk_idx, axis=1)


def kernel(a, b, W1a, b1a, W1b, b1b, W2a, b2a, W2b, b2b):
    S1 = jnp.concatenate([a, b], axis=1)
    fused1 = _select_topk_by_entropy(S1)
    S2 = jnp.concatenate([fused1, b], axis=1)
    fused2 = _select_topk_by_entropy(S2)
    logits1, logits2 = _mlp_heads(
        fused1, fused2, W1a, b1a, W1b, b1b, W2a, b2a, W2b, b2b
    )
    return (logits1, logits2, fused1, fused2)
